# baseline (speedup 1.0000x reference)
import functools
import math

import jax
import jax.numpy as jnp
from jax import lax
from jax.experimental import pallas as pl
from jax.experimental.pallas import tpu as pltpu
from jax.experimental.pallas import tpu_sc as plsc

K = [5, 64, 128, 128, 128]
RES = [50000, 35000, 20000, 10000, 5000]
NB = 2
EPS = 1e-5
GROUPS = 16
NWORKERS = 32


def _pcall(*a, **kw):
    return pl.pallas_call(*a, **kw)


def _gather_rows(table, idx):
    T, C = table.shape
    (M,) = idx.shape
    chunk = 448
    stride = 2 * chunk
    per_w = -(-M // NWORKERS)
    per_w = -(-per_w // stride) * stride
    m_pad = per_w * NWORKERS
    pairs = per_w // stride
    if m_pad != M:
        idx = jnp.concatenate([idx, jnp.zeros((m_pad - M,), jnp.int32)])

    mesh = plsc.VectorSubcoreMesh(core_axis_name="c", subcore_axis_name="s")

    @functools.partial(
        pl.kernel,
        mesh=mesh,
        out_type=jax.ShapeDtypeStruct((m_pad, C), jnp.float32),
        scratch_types=[
            pltpu.VMEM((chunk,), jnp.int32),
            pltpu.VMEM((chunk,), jnp.int32),
            pltpu.VMEM((chunk, C), jnp.float32),
            pltpu.VMEM((chunk, C), jnp.float32),
            pltpu.SemaphoreType.DMA,
            pltpu.SemaphoreType.DMA,
            pltpu.SemaphoreType.DMA,
            pltpu.SemaphoreType.DMA,
        ],
    )
    def gk(table_hbm, idx_hbm, out_hbm, idx_a, idx_b, rows_a, rows_b,
           gsem_a, gsem_b, ssem_a, ssem_b):
        wid = lax.axis_index("s") * 2 + lax.axis_index("c")
        base = wid * per_w

        def start(j, idx_v, rows_v, gsem):
            pltpu.sync_copy(idx_hbm.at[pl.ds(base + j * chunk, chunk)], idx_v)
            pltpu.async_copy(table_hbm.at[idx_v], rows_v, gsem)

        def drain(j, idx_v, rows_v, gsem, ssem):
            pltpu.make_async_copy(table_hbm.at[idx_v], rows_v, gsem).wait()
            pltpu.async_copy(rows_v, out_hbm.at[pl.ds(base + j * chunk, chunk)], ssem)

        def wait_store(rows_v, ssem):
            pltpu.make_async_copy(rows_v, out_hbm.at[pl.ds(base, chunk)], ssem).wait()

        start(0, idx_a, rows_a, gsem_a)

        def body(p, carry):
            j = 2 * p

            @pl.when(p > 0)
            def _():
                wait_store(rows_b, ssem_b)

            start(j + 1, idx_b, rows_b, gsem_b)
            drain(j, idx_a, rows_a, gsem_a, ssem_a)

            @pl.when(p + 1 < pairs)
            def _():
                wait_store(rows_a, ssem_a)
                start(j + 2, idx_a, rows_a, gsem_a)

            drain(j + 1, idx_b, rows_b, gsem_b, ssem_b)
            return carry

        lax.fori_loop(0, pairs, body, 0)
        wait_store(rows_a, ssem_a)
        wait_store(rows_b, ssem_b)

    out = gk(table, idx)
    return out[:M] if m_pad != M else out


def _conv(x, g, wcat, bn_scale=None, bn_shift=None, res=None, block=1000):
    N, cin = x.shape
    cout = wcat.shape[1]
    R = block
    assert N % R == 0
    bn = bn_scale is not None
    has_res = res is not None

    def body(*refs):
        x_ref, g_ref, w_ref = refs[0], refs[1], refs[2]
        rest = list(refs[3:-1])
        out_ref = refs[-1]
        if bn:
            sc = rest.pop(0)[0, :]
            sh = rest.pop(0)[0, :]
            f = lambda v: jnp.maximum(v, 0.0) * sc[None, :] + sh[None, :]
        else:
            f = lambda v: v
        xb = f(x_ref[...])
        a = f(g_ref[0])
        b = f(g_ref[1])
        c = f(g_ref[2])
        d = f(g_ref[3])
        taps = jnp.concatenate(
            [xb, a + c, b + d, jnp.abs(a - c), jnp.abs(b - d)], axis=1
        )
        acc = jnp.dot(taps, w_ref[...], preferred_element_type=jnp.float32)
        if has_res:
            acc = jnp.maximum(acc + rest.pop(0)[...], 0.0)
        out_ref[...] = acc

    in_specs = [
        pl.BlockSpec((R, cin), lambda i: (i, 0)),
        pl.BlockSpec((4, R, cin), lambda i: (0, i, 0)),
        pl.BlockSpec((5 * cin, cout), lambda i: (0, 0)),
    ]
    args = [x, g, wcat]
    if bn:
        in_specs += [
            pl.BlockSpec((1, cin), lambda i: (0, 0)),
            pl.BlockSpec((1, cin), lambda i: (0, 0)),
        ]
        args += [bn_scale[None, :], bn_shift[None, :]]
    if has_res:
        in_specs.append(pl.BlockSpec((R, cout), lambda i: (i, 0)))
        args.append(res)
    return _pcall(
        body,
        grid=(N // R,),
        in_specs=in_specs,
        out_specs=pl.BlockSpec((R, cout), lambda i: (i, 0)),
        out_shape=jax.ShapeDtypeStruct((N, cout), jnp.float32),
    )(*args)


def _bn_stats(y, block=1000):
    N, C = y.shape
    R = block

    def body(y_ref, out_ref):
        i = pl.program_id(0)
        t = jnp.maximum(y_ref[...], 0.0)
        blk = jnp.stack([jnp.sum(t, axis=0), jnp.sum(t * t, axis=0)], axis=0)

        @pl.when(i == 0)
        def _():
            out_ref[...] = blk

        @pl.when(i > 0)
        def _():
            out_ref[...] += blk

    return _pcall(
        body,
        grid=(N // R,),
        in_specs=[pl.BlockSpec((R, C), lambda i: (i, 0))],
        out_specs=pl.BlockSpec((2, C), lambda i: (0, 0)),
        out_shape=jax.ShapeDtypeStruct((2, C), jnp.float32),
    )(y)


def _gn_stats(x, E, block=1000):
    N, C = x.shape
    R = block
    J = E // R

    def body(x_ref, out_ref):
        j = pl.program_id(1)
        xb = x_ref[...]
        blk = jnp.stack([jnp.sum(xb, axis=0), jnp.sum(xb * xb, axis=0)], axis=0)

        @pl.when(j == 0)
        def _():
            out_ref[...] = blk[None]

        @pl.when(j > 0)
        def _():
            out_ref[...] += blk[None]

    return _pcall(
        body,
        grid=(NB, J),
        in_specs=[pl.BlockSpec((R, C), lambda b, j: (b * J + j, 0))],
        out_specs=pl.BlockSpec((1, 2, C), lambda b, j: (b, 0, 0)),
        out_shape=jax.ShapeDtypeStruct((NB, 2, C), jnp.float32),
    )(x)


def _gn_apply(x, gscale, gshift, E, block=1000):
    N, C = x.shape
    R = block
    J = E // R

    def body(x_ref, sc_ref, sh_ref, z_ref, n_ref):
        z = jnp.maximum(x_ref[...] * sc_ref[0, 0][None, :] + sh_ref[0, 0][None, :], 0.0)
        z_ref[...] = z
        n_ref[...] = jnp.sum(z * z, axis=1, keepdims=True)

    z, norms = _pcall(
        body,
        grid=(NB, J),
        in_specs=[
            pl.BlockSpec((R, C), lambda b, j: (b * J + j, 0)),
            pl.BlockSpec((1, 1, C), lambda b, j: (b, 0, 0)),
            pl.BlockSpec((1, 1, C), lambda b, j: (b, 0, 0)),
        ],
        out_specs=[
            pl.BlockSpec((R, C), lambda b, j: (b * J + j, 0)),
            pl.BlockSpec((R, 1), lambda b, j: (b * J + j, 0)),
        ],
        out_shape=[
            jax.ShapeDtypeStruct((N, C), jnp.float32),
            jax.ShapeDtypeStruct((N, 1), jnp.float32),
        ],
    )(x, gscale[:, None, :], gshift[:, None, :])
    return z, norms.reshape(NB, E)


def _mean_fc(x, E, w1, b1, w2, b2):
    N, C = x.shape

    def mbody(x_ref, out_ref):
        out_ref[...] = jnp.mean(x_ref[...], axis=0)[None, None]

    xm = _pcall(
        mbody,
        grid=(NB,),
        in_specs=[pl.BlockSpec((E, C), lambda b: (b, 0))],
        out_specs=pl.BlockSpec((1, 1, C), lambda b: (b, 0, 0)),
        out_shape=jax.ShapeDtypeStruct((NB, 1, C), jnp.float32),
    )(x).reshape(NB, C)

    f, ncls = w1.shape[1], w2.shape[1]

    def fcbody(x_ref, w1_ref, b1_ref, w2_ref, b2_ref, out_ref):
        h = jnp.maximum(
            jnp.dot(x_ref[...], w1_ref[...], preferred_element_type=jnp.float32)
            + b1_ref[...],
            0.0,
        )
        out_ref[...] = (
            jnp.dot(h, w2_ref[...], preferred_element_type=jnp.float32) + b2_ref[...]
        )

    return _pcall(
        fcbody,
        out_shape=jax.ShapeDtypeStruct((NB, ncls), jnp.float32),
    )(xm, w1, b1[None, :], w2, b2[None, :])


def _pool_threshold(bits, target):
    def body(b_ref, o_ref):
        bb = b_ref[...]
        lo = jnp.zeros((NB, 1), jnp.int32)
        for k in range(30, -1, -1):
            cand = lo + (1 << k)
            cnt = jnp.sum((bb >= cand).astype(jnp.int32), axis=1, keepdims=True)
            lo = jnp.where(cnt >= target, cand, lo)
        cntgt = jnp.sum((bb > lo).astype(jnp.int32), axis=1, keepdims=True)
        r = target - cntgt
        o_ref[...] = jnp.concatenate(
            [lo, r, jnp.zeros((NB, 14), jnp.int32)], axis=1
        )

    return _pcall(
        body,
        out_shape=jax.ShapeDtypeStruct((NB, 16), jnp.int32),
    )(bits)


def _pool_sc(bits_pad, thr_pad, gemm_t, E, target):
    nwb = 16
    cE = -(-E // (nwb * 16)) * 16
    Ep = nwb * cE
    Tp = -(-target // 256) * 256
    opw = Tp // nwb
    nsc = cE // 16

    mesh = plsc.VectorSubcoreMesh(core_axis_name="c", subcore_axis_name="s")

    @functools.partial(
        pl.kernel,
        mesh=mesh,
        out_type=(
            jax.ShapeDtypeStruct((NB * Tp,), jnp.int32),
            jax.ShapeDtypeStruct((NB * 4 * Tp,), jnp.int32),
        ),
        compiler_params=pltpu.CompilerParams(needs_layout_passes=False),
        scratch_types=[
            pltpu.VMEM((Ep,), jnp.int32),
            pltpu.VMEM((Ep,), jnp.int32),
            pltpu.VMEM((cE,), jnp.int32),
            pltpu.VMEM((cE + 16,), jnp.int32),
            pltpu.VMEM((cE,), jnp.int32),
            pltpu.VMEM((opw,), jnp.int32),
            pltpu.VMEM((opw,), jnp.int32),
            pltpu.VMEM((nwb * 16,), jnp.int32),
            pltpu.VMEM((16,), jnp.int32),
            pltpu.VMEM((16,), jnp.int32),
            pltpu.VMEM((16,), jnp.int32),
            pltpu.VMEM((16,), jnp.int32),
            pltpu.VMEM_SHARED((Ep,), jnp.int32),
            pltpu.VMEM_SHARED((Ep,), jnp.int32),
            pltpu.VMEM_SHARED((nwb * 16,), jnp.int32),
        ],
    )
    def pk(bits_hbm, thr_hbm, gt_hbm, keep_hbm, ng_hbm,
           bigA, bigB, nb_c, compact_loc, m_c, keep_loc, out_loc,
           cnt_loc, thr_loc, cv_loc, spfx_loc, tmp_loc, sp_sh, m_sh, cnt_sh):
        b = lax.axis_index("c")
        w = lax.axis_index("s")
        iota = lax.iota(jnp.int32, 16)
        zeros16 = jnp.zeros((16,), jnp.int32)

        def vgather(vec, idx):
            return vec.at[idx].get(mode="promise_in_bounds")

        def icumsum(v):
            acc = v
            for k in (1, 2, 4, 8):
                g = vgather(acc, jnp.clip(iota - k, 0, 15))
                acc = acc + jnp.where(iota >= k, g, 0)
            return acc

        def splat(vec, i):
            return vgather(vec, jnp.full((16,), i, jnp.int32))

        def total_splat(v):
            return splat(icumsum(v), 15)

        pltpu.sync_copy(bits_hbm.at[pl.ds(b * Ep + w * cE, cE)], nb_c)
        pltpu.sync_copy(thr_hbm.at[pl.ds(b * 16, 16)], thr_loc)
        t_v = splat(thr_loc[...], 0)
        r_v = splat(thr_loc[...], 1)

        def scA(j, carry):
            gtc, eqc = carry
            v = nb_c[pl.ds(j * 16, 16)]
            gtc = gtc + (v > t_v).astype(jnp.int32)
            eqc = eqc + (v == t_v).astype(jnp.int32)
            return gtc, eqc

        z0 = jnp.zeros((), jnp.int32)
        gtc, eqc = lax.fori_loop(0, nsc, scA, (zeros16, zeros16))
        gt_tot = total_splat(gtc)
        eq_tot = total_splat(eqc)
        cv_loc[...] = jnp.where(iota == 0, gt_tot, jnp.where(iota == 1, eq_tot, 0))
        pltpu.sync_copy(cv_loc, cnt_sh.at[pl.ds(w * 16, 16)])
        plsc.subcore_barrier()

        pltpu.sync_copy(cnt_sh, cnt_loc)
        gt_all = plsc.load_gather(cnt_loc, [iota * 16])
        eq_all = plsc.load_gather(cnt_loc, [iota * 16 + 1])
        eqpfx = icumsum(eq_all) - eq_all
        take = jnp.clip(r_v - eqpfx, 0, eq_all)
        sel_all = gt_all + take
        spfx = icumsum(sel_all) - sel_all
        my_pfx_v = splat(spfx, w)
        my_take_v = splat(take, w)

        def zm(j, c):
            m_c[pl.ds(j * 16, 16)] = zeros16
            return c

        lax.fori_loop(0, nsc, zm, z0)

        def scC(j, carry):
            pos_v, eqr_v = carry
            v = nb_c[pl.ds(j * 16, 16)]
            gt = v > t_v
            eq = v == t_v
            eqi = eq.astype(jnp.int32)
            eqinc = icumsum(eqi)
            eqtot = splat(eqinc, 15)
            eqrank = eqr_v + eqinc - eqi
            sel = jnp.logical_or(gt, jnp.logical_and(eq, eqrank < my_take_v))
            seli = sel.astype(jnp.int32)
            selinc = icumsum(seli)
            seltot = splat(selinc, 15)
            posv = pos_v + selinc - seli
            lidx = j * 16 + iota
            plsc.store_scatter(m_c, [lidx], posv + 1, mask=sel)
            plsc.store_scatter(
                compact_loc, [posv - my_pfx_v], w * cE + lidx, mask=sel
            )
            return pos_v + seltot, eqr_v + eqtot

        lax.fori_loop(0, nsc, scC, (my_pfx_v, zeros16))
        pltpu.sync_copy(compact_loc.at[pl.ds(0, cE)], sp_sh.at[pl.ds(w * cE, cE)])
        pltpu.sync_copy(m_c, m_sh.at[pl.ds(w * cE, cE)])
        plsc.subcore_barrier()

        pltpu.sync_copy(sp_sh, bigA)
        sp_vecs = [splat(spfx, kk) for kk in range(nwb)]

        def scK(j, c):
            p_v = w * opw + j * 16 + iota
            jv = jnp.full((16,), -1, jnp.int32)
            for kk in range(nwb):
                jv = jv + (p_v >= sp_vecs[kk]).astype(jnp.int32)
            jv = jnp.clip(jv, 0, nwb - 1)
            spjv = vgather(spfx, jv)
            src = jnp.clip(jv * cE + (p_v - spjv), 0, Ep - 1)
            keep_loc[pl.ds(j * 16, 16)] = plsc.load_gather(bigA, [src])
            return c

        lax.fori_loop(0, opw // 16, scK, z0)
        pltpu.sync_copy(keep_loc, keep_hbm.at[pl.ds(b * Tp + w * opw, opw)])

        pltpu.sync_copy(m_sh, bigA)

        for k in range(4):
            pltpu.sync_copy(gt_hbm.at[pl.ds((b * 4 + k) * Ep, Ep)], bigB)

            def scD(j, c):
                p_v = w * opw + j * 16 + iota
                kv = jnp.clip(keep_loc[pl.ds(j * 16, 16)], 0, Ep - 1)
                gvals = plsc.load_gather(bigB, [kv])
                ng1 = plsc.load_gather(bigA, [jnp.clip(gvals, 0, Ep - 1)])
                out_loc[pl.ds(j * 16, 16)] = jnp.where(ng1 == 0, p_v, ng1 - 1)
                return c

            lax.fori_loop(0, opw // 16, scD, z0)
            pltpu.sync_copy(
                out_loc, ng_hbm.at[pl.ds((b * 4 + k) * Tp + w * opw, opw)]
            )

    return pk(bits_pad, thr_pad, gemm_t), Tp


def _mesh_pool(norms, gemm, E, target):
    nwb = 16
    cE = -(-E // (nwb * 16)) * 16
    Ep = nwb * cE
    bits = lax.bitcast_convert_type(norms, jnp.int32)
    bits = jnp.pad(bits, ((0, 0), (0, Ep - E)), constant_values=-1)
    thr = _pool_threshold(bits, target)
    gemm_t = jnp.pad(
        jnp.transpose(gemm, (0, 2, 1)), ((0, 0), (0, 0), (0, Ep - E))
    ).reshape(-1)
    (keep, ng), Tp = _pool_sc(bits.reshape(-1), thr.reshape(-1), gemm_t, E, target)
    keep = keep.reshape(NB, Tp)[:, :target]
    ng = ng.reshape(NB, 4, Tp)[:, :, :target].transpose(0, 2, 1)
    return keep, ng


def _neighbor_idx(gemm, E):
    boff = (jnp.arange(NB, dtype=jnp.int32) * E)[None, :, None]
    idx = gemm.astype(jnp.int32).transpose(2, 0, 1) + boff
    return idx.reshape(-1)


CP = 128


def _wcat_p(w):
    cout, cin, _ = w.shape
    wt = jnp.transpose(w, (2, 1, 0))
    wt = jnp.pad(wt, ((0, 0), (0, CP - cin), (0, CP - cout)))
    return wt.reshape(5 * CP, CP)


def _padc(v):
    return jnp.pad(v, (0, CP - v.shape[0]))


def kernel(x, gemm_edges, feature_values, params):
    del feature_values
    B, C0, E = x.shape
    feat = jnp.transpose(x, (0, 2, 1)).reshape(B * E, C0)
    feat = jnp.concatenate([feat, jnp.zeros((B * E, CP - C0), jnp.float32)], axis=1)
    gemm = gemm_edges.astype(jnp.int32)

    for i in range(4):
        blk = params["block%d" % i]
        E = RES[i]
        N = B * E
        cout = K[i + 1]
        w0 = _wcat_p(blk["w0"])
        w1 = _wcat_p(blk["w1"])

        nidx = _neighbor_idx(gemm, E)
        g0 = _gather_rows(feat, nidx).reshape(4, N, CP)
        y = _conv(feat, g0, w0)

        st = _bn_stats(y)
        mean = st[0] / N
        var = st[1] / N - mean * mean
        bscale = _padc(blk["bn_g1"]) * lax.rsqrt(var + EPS)
        bshift = _padc(blk["bn_b1"]) - mean * bscale

        g1 = _gather_rows(y, nidx).reshape(4, N, CP)
        x2 = _conv(y, g1, w1, bn_scale=bscale, bn_shift=bshift, res=y)

        gst = _gn_stats(x2, E)
        cg = cout // GROUPS
        gs = gst[:, :, :cout].reshape(NB, 2, GROUPS, cg).sum(axis=3)
        gm = gs[:, 0] / (cg * E)
        gv = gs[:, 1] / (cg * E) - gm * gm
        grs = lax.rsqrt(gv + EPS)
        gscale = blk["gn_g"][None, :] * jnp.repeat(grs, cg, axis=1)
        gshift = blk["gn_b"][None, :] - jnp.repeat(gm, cg, axis=1) * gscale
        gscale = jnp.pad(gscale, ((0, 0), (0, CP - cout)))
        gshift = jnp.pad(gshift, ((0, 0), (0, CP - cout)))

        z, norms = _gn_apply(x2, gscale, gshift, E)

        target = RES[i + 1]
        keep, gemm = _mesh_pool(norms, gemm, E, target)
        kflat = (keep + (jnp.arange(NB, dtype=jnp.int32) * E)[:, None]).reshape(-1)
        feat = _gather_rows(z, kflat)

    return _mean_fc(
        feat,
        RES[4],
        params["fc1_w"].T,
        params["fc1_b"],
        params["fc2_w"].T,
        params["fc2_b"],
    )


# speedup vs baseline: 1.2519x; 1.2519x over previous
import functools
import math

import jax
import jax.numpy as jnp
from jax import lax
from jax.experimental import pallas as pl
from jax.experimental.pallas import tpu as pltpu
from jax.experimental.pallas import tpu_sc as plsc

K = [5, 64, 128, 128, 128]
RES = [50000, 35000, 20000, 10000, 5000]
NB = 2
EPS = 1e-5
GROUPS = 16
NWORKERS = 32


def _pcall(*a, **kw):
    return pl.pallas_call(*a, **kw)


def _gather_rows(table, idx):
    T, C = table.shape
    (M,) = idx.shape
    chunk = 192
    stride = 2 * chunk
    per_w = -(-M // NWORKERS)
    per_w = -(-per_w // stride) * stride
    m_pad = per_w * NWORKERS
    pairs = per_w // stride
    if m_pad != M:
        idx = jnp.concatenate([idx, jnp.zeros((m_pad - M,), jnp.int32)])

    mesh = plsc.VectorSubcoreMesh(core_axis_name="c", subcore_axis_name="s")

    @functools.partial(
        pl.kernel,
        mesh=mesh,
        out_type=jax.ShapeDtypeStruct((m_pad, C), jnp.float32),
        scratch_types=[
            pltpu.VMEM((chunk,), jnp.int32),
            pltpu.VMEM((chunk,), jnp.int32),
            pltpu.VMEM((chunk, C), jnp.float32),
            pltpu.VMEM((chunk, C), jnp.float32),
            pltpu.SemaphoreType.DMA,
            pltpu.SemaphoreType.DMA,
            pltpu.SemaphoreType.DMA,
            pltpu.SemaphoreType.DMA,
        ],
    )
    def gk(table_hbm, idx_hbm, out_hbm, idx_a, idx_b, rows_a, rows_b,
           gsem_a, gsem_b, ssem_a, ssem_b):
        wid = lax.axis_index("s") * 2 + lax.axis_index("c")
        base = wid * per_w

        def start(j, idx_v, rows_v, gsem):
            pltpu.sync_copy(idx_hbm.at[pl.ds(base + j * chunk, chunk)], idx_v)
            pltpu.async_copy(table_hbm.at[idx_v], rows_v, gsem)

        def drain(j, idx_v, rows_v, gsem, ssem):
            pltpu.make_async_copy(table_hbm.at[idx_v], rows_v, gsem).wait()
            pltpu.async_copy(rows_v, out_hbm.at[pl.ds(base + j * chunk, chunk)], ssem)

        def wait_store(rows_v, ssem):
            pltpu.make_async_copy(rows_v, out_hbm.at[pl.ds(base, chunk)], ssem).wait()

        start(0, idx_a, rows_a, gsem_a)

        def body(p, carry):
            j = 2 * p

            @pl.when(p > 0)
            def _():
                wait_store(rows_b, ssem_b)

            start(j + 1, idx_b, rows_b, gsem_b)
            drain(j, idx_a, rows_a, gsem_a, ssem_a)

            @pl.when(p + 1 < pairs)
            def _():
                wait_store(rows_a, ssem_a)
                start(j + 2, idx_a, rows_a, gsem_a)

            drain(j + 1, idx_b, rows_b, gsem_b, ssem_b)
            return carry

        lax.fori_loop(0, pairs, body, 0)
        wait_store(rows_a, ssem_a)
        wait_store(rows_b, ssem_b)

    out = gk(table, idx)
    return out[:M] if m_pad != M else out


def _conv(x, g, wcat, bn_scale=None, bn_shift=None, res=None, block=1000):
    N, cin = x.shape
    cout = wcat.shape[1]
    R = block
    assert N % R == 0
    bn = bn_scale is not None
    has_res = res is not None

    def body(*refs):
        x_ref, g_ref, w_ref = refs[0], refs[1], refs[2]
        rest = list(refs[3:-1])
        out_ref = refs[-1]
        if bn:
            sc = rest.pop(0)[0, :]
            sh = rest.pop(0)[0, :]
            f = lambda v: jnp.maximum(v, 0.0) * sc[None, :] + sh[None, :]
        else:
            f = lambda v: v
        xb = f(x_ref[...])
        a = f(g_ref[0])
        b = f(g_ref[1])
        c = f(g_ref[2])
        d = f(g_ref[3])
        taps = jnp.concatenate(
            [xb, a + c, b + d, jnp.abs(a - c), jnp.abs(b - d)], axis=1
        )
        acc = jnp.dot(taps, w_ref[...], preferred_element_type=jnp.float32)
        if has_res:
            acc = jnp.maximum(acc + rest.pop(0)[...], 0.0)
        out_ref[...] = acc

    in_specs = [
        pl.BlockSpec((R, cin), lambda i: (i, 0)),
        pl.BlockSpec((4, R, cin), lambda i: (0, i, 0)),
        pl.BlockSpec((5 * cin, cout), lambda i: (0, 0)),
    ]
    args = [x, g, wcat]
    if bn:
        in_specs += [
            pl.BlockSpec((1, cin), lambda i: (0, 0)),
            pl.BlockSpec((1, cin), lambda i: (0, 0)),
        ]
        args += [bn_scale[None, :], bn_shift[None, :]]
    if has_res:
        in_specs.append(pl.BlockSpec((R, cout), lambda i: (i, 0)))
        args.append(res)
    return _pcall(
        body,
        grid=(N // R,),
        in_specs=in_specs,
        out_specs=pl.BlockSpec((R, cout), lambda i: (i, 0)),
        out_shape=jax.ShapeDtypeStruct((N, cout), jnp.float32),
    )(*args)


def _bn_stats(y, block=1000):
    N, C = y.shape
    R = block

    def body(y_ref, out_ref):
        i = pl.program_id(0)
        t = jnp.maximum(y_ref[...], 0.0)
        blk = jnp.stack([jnp.sum(t, axis=0), jnp.sum(t * t, axis=0)], axis=0)

        @pl.when(i == 0)
        def _():
            out_ref[...] = blk

        @pl.when(i > 0)
        def _():
            out_ref[...] += blk

    return _pcall(
        body,
        grid=(N // R,),
        in_specs=[pl.BlockSpec((R, C), lambda i: (i, 0))],
        out_specs=pl.BlockSpec((2, C), lambda i: (0, 0)),
        out_shape=jax.ShapeDtypeStruct((2, C), jnp.float32),
    )(y)


def _gn_stats(x, E, block=1000):
    N, C = x.shape
    R = block
    J = E // R

    def body(x_ref, out_ref):
        j = pl.program_id(1)
        xb = x_ref[...]
        blk = jnp.stack([jnp.sum(xb, axis=0), jnp.sum(xb * xb, axis=0)], axis=0)

        @pl.when(j == 0)
        def _():
            out_ref[...] = blk[None]

        @pl.when(j > 0)
        def _():
            out_ref[...] += blk[None]

    return _pcall(
        body,
        grid=(NB, J),
        in_specs=[pl.BlockSpec((R, C), lambda b, j: (b * J + j, 0))],
        out_specs=pl.BlockSpec((1, 2, C), lambda b, j: (b, 0, 0)),
        out_shape=jax.ShapeDtypeStruct((NB, 2, C), jnp.float32),
    )(x)


def _gn_apply(x, gscale, gshift, E, block=1000):
    N, C = x.shape
    R = block
    J = E // R

    def body(x_ref, sc_ref, sh_ref, z_ref, n_ref):
        z = jnp.maximum(x_ref[...] * sc_ref[0, 0][None, :] + sh_ref[0, 0][None, :], 0.0)
        z_ref[...] = z
        n_ref[...] = jnp.sum(z * z, axis=1, keepdims=True)

    z, norms = _pcall(
        body,
        grid=(NB, J),
        in_specs=[
            pl.BlockSpec((R, C), lambda b, j: (b * J + j, 0)),
            pl.BlockSpec((1, 1, C), lambda b, j: (b, 0, 0)),
            pl.BlockSpec((1, 1, C), lambda b, j: (b, 0, 0)),
        ],
        out_specs=[
            pl.BlockSpec((R, C), lambda b, j: (b * J + j, 0)),
            pl.BlockSpec((R, 1), lambda b, j: (b * J + j, 0)),
        ],
        out_shape=[
            jax.ShapeDtypeStruct((N, C), jnp.float32),
            jax.ShapeDtypeStruct((N, 1), jnp.float32),
        ],
    )(x, gscale[:, None, :], gshift[:, None, :])
    return z, norms.reshape(NB, E)


def _mean_fc(x, E, w1, b1, w2, b2):
    N, C = x.shape

    def mbody(x_ref, out_ref):
        out_ref[...] = jnp.mean(x_ref[...], axis=0)[None, None]

    xm = _pcall(
        mbody,
        grid=(NB,),
        in_specs=[pl.BlockSpec((E, C), lambda b: (b, 0))],
        out_specs=pl.BlockSpec((1, 1, C), lambda b: (b, 0, 0)),
        out_shape=jax.ShapeDtypeStruct((NB, 1, C), jnp.float32),
    )(x).reshape(NB, C)

    f, ncls = w1.shape[1], w2.shape[1]

    def fcbody(x_ref, w1_ref, b1_ref, w2_ref, b2_ref, out_ref):
        h = jnp.maximum(
            jnp.dot(x_ref[...], w1_ref[...], preferred_element_type=jnp.float32)
            + b1_ref[...],
            0.0,
        )
        out_ref[...] = (
            jnp.dot(h, w2_ref[...], preferred_element_type=jnp.float32) + b2_ref[...]
        )

    return _pcall(
        fcbody,
        out_shape=jax.ShapeDtypeStruct((NB, ncls), jnp.float32),
    )(xm, w1, b1[None, :], w2, b2[None, :])


def _pool_threshold(bits, target):
    def body(b_ref, o_ref):
        bb = b_ref[...]
        lo = jnp.zeros((NB, 1), jnp.int32)
        for k in range(30, -1, -1):
            cand = lo + (1 << k)
            cnt = jnp.sum((bb >= cand).astype(jnp.int32), axis=1, keepdims=True)
            lo = jnp.where(cnt >= target, cand, lo)
        cntgt = jnp.sum((bb > lo).astype(jnp.int32), axis=1, keepdims=True)
        r = target - cntgt
        o_ref[...] = jnp.concatenate(
            [lo, r, jnp.zeros((NB, 14), jnp.int32)], axis=1
        )

    return _pcall(
        body,
        out_shape=jax.ShapeDtypeStruct((NB, 16), jnp.int32),
    )(bits)


def _pool_sc(bits_pad, thr_pad, gemm_t, E, target):
    nwb = 16
    cE = -(-E // (nwb * 16)) * 16
    Ep = nwb * cE
    Tp = -(-target // 256) * 256
    opw = Tp // nwb
    nsc = cE // 16

    mesh = plsc.VectorSubcoreMesh(core_axis_name="c", subcore_axis_name="s")

    @functools.partial(
        pl.kernel,
        mesh=mesh,
        out_type=(
            jax.ShapeDtypeStruct((NB * Tp,), jnp.int32),
            jax.ShapeDtypeStruct((NB * 4 * Tp,), jnp.int32),
        ),
        compiler_params=pltpu.CompilerParams(needs_layout_passes=False),
        scratch_types=[
            pltpu.VMEM((Ep,), jnp.int32),
            pltpu.VMEM((Ep,), jnp.int32),
            pltpu.VMEM((cE,), jnp.int32),
            pltpu.VMEM((cE + 16,), jnp.int32),
            pltpu.VMEM((cE,), jnp.int32),
            pltpu.VMEM((opw,), jnp.int32),
            pltpu.VMEM((opw,), jnp.int32),
            pltpu.VMEM((nwb * 16,), jnp.int32),
            pltpu.VMEM((16,), jnp.int32),
            pltpu.VMEM((16,), jnp.int32),
            pltpu.VMEM((16,), jnp.int32),
            pltpu.VMEM((16,), jnp.int32),
            pltpu.VMEM_SHARED((Ep,), jnp.int32),
            pltpu.VMEM_SHARED((Ep,), jnp.int32),
            pltpu.VMEM_SHARED((nwb * 16,), jnp.int32),
        ],
    )
    def pk(bits_hbm, thr_hbm, gt_hbm, keep_hbm, ng_hbm,
           bigA, bigB, nb_c, compact_loc, m_c, keep_loc, out_loc,
           cnt_loc, thr_loc, cv_loc, spfx_loc, tmp_loc, sp_sh, m_sh, cnt_sh):
        b = lax.axis_index("c")
        w = lax.axis_index("s")
        iota = lax.iota(jnp.int32, 16)
        zeros16 = jnp.zeros((16,), jnp.int32)

        def vgather(vec, idx):
            return vec.at[idx].get(mode="promise_in_bounds")

        def icumsum(v):
            acc = v
            for k in (1, 2, 4, 8):
                g = vgather(acc, jnp.clip(iota - k, 0, 15))
                acc = acc + jnp.where(iota >= k, g, 0)
            return acc

        def splat(vec, i):
            return vgather(vec, jnp.full((16,), i, jnp.int32))

        def total_splat(v):
            return splat(icumsum(v), 15)

        pltpu.sync_copy(bits_hbm.at[pl.ds(b * Ep + w * cE, cE)], nb_c)
        pltpu.sync_copy(thr_hbm.at[pl.ds(b * 16, 16)], thr_loc)
        t_v = splat(thr_loc[...], 0)
        r_v = splat(thr_loc[...], 1)

        def scA(j, carry):
            gtc, eqc = carry
            v = nb_c[pl.ds(j * 16, 16)]
            gtc = gtc + (v > t_v).astype(jnp.int32)
            eqc = eqc + (v == t_v).astype(jnp.int32)
            return gtc, eqc

        z0 = jnp.zeros((), jnp.int32)
        gtc, eqc = lax.fori_loop(0, nsc, scA, (zeros16, zeros16))
        gt_tot = total_splat(gtc)
        eq_tot = total_splat(eqc)
        cv_loc[...] = jnp.where(iota == 0, gt_tot, jnp.where(iota == 1, eq_tot, 0))
        pltpu.sync_copy(cv_loc, cnt_sh.at[pl.ds(w * 16, 16)])
        plsc.subcore_barrier()

        pltpu.sync_copy(cnt_sh, cnt_loc)
        gt_all = plsc.load_gather(cnt_loc, [iota * 16])
        eq_all = plsc.load_gather(cnt_loc, [iota * 16 + 1])
        eqpfx = icumsum(eq_all) - eq_all
        take = jnp.clip(r_v - eqpfx, 0, eq_all)
        sel_all = gt_all + take
        spfx = icumsum(sel_all) - sel_all
        my_pfx_v = splat(spfx, w)
        my_take_v = splat(take, w)

        def zm(j, c):
            m_c[pl.ds(j * 16, 16)] = zeros16
            return c

        lax.fori_loop(0, nsc, zm, z0)

        def scC(j, carry):
            pos_v, eqr_v = carry
            v = nb_c[pl.ds(j * 16, 16)]
            gt = v > t_v
            eq = v == t_v
            eqi = eq.astype(jnp.int32)
            eqinc = icumsum(eqi)
            eqtot = splat(eqinc, 15)
            eqrank = eqr_v + eqinc - eqi
            sel = jnp.logical_or(gt, jnp.logical_and(eq, eqrank < my_take_v))
            seli = sel.astype(jnp.int32)
            selinc = icumsum(seli)
            seltot = splat(selinc, 15)
            posv = pos_v + selinc - seli
            lidx = j * 16 + iota
            plsc.store_scatter(m_c, [lidx], posv + 1, mask=sel)
            plsc.store_scatter(
                compact_loc, [posv - my_pfx_v], w * cE + lidx, mask=sel
            )
            return pos_v + seltot, eqr_v + eqtot

        lax.fori_loop(0, nsc, scC, (my_pfx_v, zeros16))
        pltpu.sync_copy(compact_loc.at[pl.ds(0, cE)], sp_sh.at[pl.ds(w * cE, cE)])
        pltpu.sync_copy(m_c, m_sh.at[pl.ds(w * cE, cE)])
        plsc.subcore_barrier()

        pltpu.sync_copy(sp_sh, bigA)
        sp_vecs = [splat(spfx, kk) for kk in range(nwb)]

        def scK(j, c):
            p_v = w * opw + j * 16 + iota
            jv = jnp.full((16,), -1, jnp.int32)
            for kk in range(nwb):
                jv = jv + (p_v >= sp_vecs[kk]).astype(jnp.int32)
            jv = jnp.clip(jv, 0, nwb - 1)
            spjv = vgather(spfx, jv)
            src = jnp.clip(jv * cE + (p_v - spjv), 0, Ep - 1)
            keep_loc[pl.ds(j * 16, 16)] = plsc.load_gather(bigA, [src])
            return c

        lax.fori_loop(0, opw // 16, scK, z0)
        pltpu.sync_copy(keep_loc, keep_hbm.at[pl.ds(b * Tp + w * opw, opw)])

        pltpu.sync_copy(m_sh, bigA)

        for k in range(4):
            pltpu.sync_copy(gt_hbm.at[pl.ds((b * 4 + k) * Ep, Ep)], bigB)

            def scD(j, c):
                p_v = w * opw + j * 16 + iota
                kv = jnp.clip(keep_loc[pl.ds(j * 16, 16)], 0, Ep - 1)
                gvals = plsc.load_gather(bigB, [kv])
                ng1 = plsc.load_gather(bigA, [jnp.clip(gvals, 0, Ep - 1)])
                out_loc[pl.ds(j * 16, 16)] = jnp.where(ng1 == 0, p_v, ng1 - 1)
                return c

            lax.fori_loop(0, opw // 16, scD, z0)
            pltpu.sync_copy(
                out_loc, ng_hbm.at[pl.ds((b * 4 + k) * Tp + w * opw, opw)]
            )

    return pk(bits_pad, thr_pad, gemm_t), Tp


def _mesh_pool(norms, gemm, E, target):
    nwb = 16
    cE = -(-E // (nwb * 16)) * 16
    Ep = nwb * cE
    bits = lax.bitcast_convert_type(norms, jnp.int32)
    bits = jnp.pad(bits, ((0, 0), (0, Ep - E)), constant_values=-1)
    thr = _pool_threshold(bits, target)
    gemm_t = jnp.pad(
        jnp.transpose(gemm, (0, 2, 1)), ((0, 0), (0, 0), (0, Ep - E))
    ).reshape(-1)
    (keep, ng), Tp = _pool_sc(bits.reshape(-1), thr.reshape(-1), gemm_t, E, target)
    keep = keep.reshape(NB, Tp)[:, :target]
    ng = ng.reshape(NB, 4, Tp)[:, :, :target].transpose(0, 2, 1)
    return keep, ng


def _neighbor_idx(gemm, E):
    boff = (jnp.arange(NB, dtype=jnp.int32) * E)[None, :, None]
    idx = gemm.astype(jnp.int32).transpose(2, 0, 1) + boff
    return idx.reshape(-1)


CP = 128


def _wcat_p(w):
    cout, cin, _ = w.shape
    wt = jnp.transpose(w, (2, 1, 0))
    wt = jnp.pad(wt, ((0, 0), (0, CP - cin), (0, CP - cout)))
    return wt.reshape(5 * CP, CP)


def _padc(v):
    return jnp.pad(v, (0, CP - v.shape[0]))


def kernel(x, gemm_edges, feature_values, params):
    del feature_values
    B, C0, E = x.shape
    feat = jnp.transpose(x, (0, 2, 1)).reshape(B * E, C0)
    feat = jnp.concatenate([feat, jnp.zeros((B * E, CP - C0), jnp.float32)], axis=1)
    gemm = gemm_edges.astype(jnp.int32)

    for i in range(4):
        blk = params["block%d" % i]
        E = RES[i]
        N = B * E
        cout = K[i + 1]
        w0 = _wcat_p(blk["w0"])
        w1 = _wcat_p(blk["w1"])

        nidx = _neighbor_idx(gemm, E)
        g0 = _gather_rows(feat, nidx).reshape(4, N, CP)
        y = _conv(feat, g0, w0)

        st = _bn_stats(y)
        mean = st[0] / N
        var = st[1] / N - mean * mean
        bscale = _padc(blk["bn_g1"]) * lax.rsqrt(var + EPS)
        bshift = _padc(blk["bn_b1"]) - mean * bscale

        g1 = _gather_rows(y, nidx).reshape(4, N, CP)
        x2 = _conv(y, g1, w1, bn_scale=bscale, bn_shift=bshift, res=y)

        gst = _gn_stats(x2, E)
        cg = cout // GROUPS
        gs = gst[:, :, :cout].reshape(NB, 2, GROUPS, cg).sum(axis=3)
        gm = gs[:, 0] / (cg * E)
        gv = gs[:, 1] / (cg * E) - gm * gm
        grs = lax.rsqrt(gv + EPS)
        gscale = blk["gn_g"][None, :] * jnp.repeat(grs, cg, axis=1)
        gshift = blk["gn_b"][None, :] - jnp.repeat(gm, cg, axis=1) * gscale
        gscale = jnp.pad(gscale, ((0, 0), (0, CP - cout)))
        gshift = jnp.pad(gshift, ((0, 0), (0, CP - cout)))

        z, norms = _gn_apply(x2, gscale, gshift, E)

        target = RES[i + 1]
        keep, gemm = _mesh_pool(norms, gemm, E, target)
        kflat = (keep + (jnp.arange(NB, dtype=jnp.int32) * E)[:, None]).reshape(-1)
        feat = _gather_rows(z, kflat)

    return _mean_fc(
        feat,
        RES[4],
        params["fc1_w"].T,
        params["fc1_b"],
        params["fc2_w"].T,
        params["fc2_b"],
    )
